# baseline (speedup 1.0000x reference)
import functools

import jax
import jax.numpy as jnp
from jax import lax
from jax.experimental import pallas as pl
from jax.experimental.pallas import tpu as pltpu
from jax.experimental.pallas import tpu_sc as plsc

B = 16384
ED = 128
_CHUNKS = ((0, 4096), (4096, 12288))


_NC = 2
_NS = 16
_NW = _NC * _NS
_IDXW = 128


def _gather_body(start, bpw, nch, ut, it, uid, iid, ts, dow, xc, et,
                 idx_u, idx_i, rows_u, rows_i, ts_v, dow_v, ebuf,
                 sem_u, sem_i, sem_s):
    wid = lax.axis_index("s") * _NC + lax.axis_index("c")
    base = wid * bpw
    hbase = start + base
    pltpu.sync_copy(uid.at[pl.ds(hbase, bpw)], idx_u)
    pltpu.sync_copy(iid.at[pl.ds(hbase, bpw)], idx_i)
    hu = [pltpu.async_copy(ut.at[idx_u.at[pl.ds(j * _IDXW, _IDXW)]],
                           rows_u.at[pl.ds(j * _IDXW, _IDXW)], sem_u)
          for j in range(nch)]
    hi = [pltpu.async_copy(it.at[idx_i.at[pl.ds(j * _IDXW, _IDXW)]],
                           rows_i.at[pl.ds(j * _IDXW, _IDXW)], sem_i)
          for j in range(nch)]
    pltpu.sync_copy(ts.at[pl.ds(hbase, bpw)], ts_v)
    pltpu.sync_copy(dow.at[pl.ds(hbase, bpw)], dow_v)
    zeros16 = jnp.zeros((16,), jnp.float32)
    ones16 = jnp.ones((16,), jnp.float32)
    for g in range(bpw // 16):
        sl = pl.ds(g * 16, 16)
        dow16 = dow_v[sl]
        ts16 = ts_v[sl]
        for j in range(16):
            if j < 7:
                val = jnp.where(dow16 == j, ones16, zeros16)
            elif j == 8:
                val = ts16
            else:
                val = zeros16
            ebuf[j, sl] = val
    hs = [pltpu.async_copy(ebuf, et.at[:, pl.ds(base, bpw)], sem_s)]
    for j in range(nch):
        hu[j].wait()
        hs.append(pltpu.async_copy(
            rows_u.at[pl.ds(j * _IDXW, _IDXW)],
            xc.at[pl.ds(base + j * _IDXW, _IDXW), pl.ds(0, ED)], sem_s))
    for j in range(nch):
        hi[j].wait()
        hs.append(pltpu.async_copy(
            rows_i.at[pl.ds(j * _IDXW, _IDXW)],
            xc.at[pl.ds(base + j * _IDXW, _IDXW), pl.ds(ED, ED)], sem_s))
    for h in hs:
        h.wait()


@functools.cache
def _make_sc_gather(start, nrows):
    bpw = nrows // _NW
    nch = bpw // _IDXW
    return pl.kernel(
        functools.partial(_gather_body, start, bpw, nch),
        out_type=(jax.ShapeDtypeStruct((nrows, 2 * ED), jnp.float32),
                  jax.ShapeDtypeStruct((16, nrows), jnp.float32)),
        mesh=plsc.VectorSubcoreMesh(core_axis_name="c", subcore_axis_name="s"),
        scratch_types=[
            pltpu.VMEM((bpw,), jnp.int32),
            pltpu.VMEM((bpw,), jnp.int32),
            pltpu.VMEM((bpw, ED), jnp.float32),
            pltpu.VMEM((bpw, ED), jnp.float32),
            pltpu.VMEM((bpw,), jnp.float32),
            pltpu.VMEM((bpw,), jnp.int32),
            pltpu.VMEM((16, bpw), jnp.float32),
            pltpu.SemaphoreType.DMA,
            pltpu.SemaphoreType.DMA,
            pltpu.SemaphoreType.DMA,
        ],
    )


_TB = 4096


def _mlp_body(xc, e, w01, w0ext, sel16,
              b0, g0, be0, m0, v0,
              w1, b1, g1, be1, m1, v1,
              w2, b2, g2, be2, m2, v2,
              wf, bf, out):
    f32 = jnp.float32
    ew = jnp.dot(sel16[...], w0ext[...], preferred_element_type=f32)

    h = jnp.dot(xc[...], w01[...], preferred_element_type=f32)
    h += lax.dot_general(e[...], ew, (((0,), (0,)), ((), ())),
                         preferred_element_type=f32)
    s = g0[...] * lax.rsqrt(v0[...] + 1e-5)
    t = (b0[...] - m0[...]) * s + be0[...]
    h = jnp.maximum(h * s + t, 0.0)

    h = jnp.dot(h, w1[...], preferred_element_type=f32)
    s = g1[...] * lax.rsqrt(v1[...] + 1e-5)
    t = (b1[...] - m1[...]) * s + be1[...]
    h = jnp.maximum(h * s + t, 0.0)

    h = jnp.dot(h, w2[...], preferred_element_type=f32)
    s = g2[...] * lax.rsqrt(v2[...] + 1e-5)
    t = (b2[...] - m2[...]) * s + be2[...]
    h = jnp.maximum(h * s + t, 0.0)

    z = jnp.dot(h, wf[...], preferred_element_type=f32)
    z8 = jnp.reshape(z, (_TB // 128, 128)) + bf[...]
    out[...] = 5.0 / (1.0 + jnp.exp(-z8))


def _full(shape):
    return pl.BlockSpec(shape, lambda i: (0, 0))


@functools.cache
def _make_mlp(nrows):
  return pl.pallas_call(
    _mlp_body,
    grid=(nrows // _TB,),
    in_specs=[
        pl.BlockSpec((_TB, 2 * ED), lambda i: (i, 0)),
        pl.BlockSpec((16, _TB), lambda i: (0, i)),
        _full((2 * ED, 1024)),
        _full((8, 1024)),
        _full((16, 8)),
        _full((1, 1024)), _full((1, 1024)), _full((1, 1024)), _full((1, 1024)), _full((1, 1024)),
        _full((1024, 512)),
        _full((1, 512)), _full((1, 512)), _full((1, 512)), _full((1, 512)), _full((1, 512)),
        _full((512, 256)),
        _full((1, 256)), _full((1, 256)), _full((1, 256)), _full((1, 256)), _full((1, 256)),
        _full((2 * ED, 1)),
        _full((1, 1)),
    ],
    out_specs=pl.BlockSpec((_TB // 128, 128), lambda i: (i, 0)),
    out_shape=jax.ShapeDtypeStruct((nrows // 128, 128), jnp.float32),
    compiler_params=pltpu.CompilerParams(
        dimension_semantics=("parallel",),
    ),
  )


def kernel(user_ids, item_ids, timestamps, day_of_week,
           user_table, item_table, day_table,
           W0, b0, g0, be0, m0, v0,
           W1, b1, g1, be1, m1, v1,
           W2, b2, g2, be2, m2, v2,
           Wf, bf):
    uid2 = user_ids.astype(jnp.int32)
    iid2 = item_ids.astype(jnp.int32)
    dow = day_of_week.astype(jnp.int32)

    w01 = W0[:2 * ED]
    w0ext = jnp.pad(W0[2 * ED:], ((0, 3), (0, 0)))
    sel16 = (jnp.zeros((16, 8), jnp.float32)
             .at[:7, 1:5].set(day_table).at[8, 0].set(1.0))

    bn = (b0.reshape(1, -1), g0.reshape(1, -1), be0.reshape(1, -1), m0.reshape(1, -1), v0.reshape(1, -1),
          W1,
          b1.reshape(1, -1), g1.reshape(1, -1), be1.reshape(1, -1), m1.reshape(1, -1), v1.reshape(1, -1),
          W2,
          b2.reshape(1, -1), g2.reshape(1, -1), be2.reshape(1, -1), m2.reshape(1, -1), v2.reshape(1, -1),
          Wf, bf.reshape(1, 1))

    outs = []
    for start, nrows in _CHUNKS:
        xc, e_c = _make_sc_gather(start, nrows)(user_table, item_table,
                                                uid2, iid2, timestamps, dow)
        outs.append(_make_mlp(nrows)(xc, e_c, w01, w0ext, sel16, *bn))
    return jnp.concatenate(outs, axis=0).reshape(B, 1)


# speedup vs baseline: 1.0373x; 1.0373x over previous
import functools

import jax
import jax.numpy as jnp
from jax import lax
from jax.experimental import pallas as pl
from jax.experimental.pallas import tpu as pltpu
from jax.experimental.pallas import tpu_sc as plsc

B = 16384
ED = 128
_CHUNKS = ((0, 8192), (8192, 8192))


_NC = 2
_NS = 16
_NW = _NC * _NS
_IDXW = 128


def _gather_body(start, bpw, nch, ut, it, uid, iid, ts, dow, xc, et,
                 idx_u, idx_i, rows_u, rows_i, ts_v, dow_v, ebuf,
                 sem_u, sem_i, sem_s):
    wid = lax.axis_index("s") * _NC + lax.axis_index("c")
    base = wid * bpw
    hbase = start + base
    pltpu.sync_copy(uid.at[pl.ds(hbase, bpw)], idx_u)
    pltpu.sync_copy(iid.at[pl.ds(hbase, bpw)], idx_i)
    hu = [pltpu.async_copy(ut.at[idx_u.at[pl.ds(j * _IDXW, _IDXW)]],
                           rows_u.at[pl.ds(j * _IDXW, _IDXW)], sem_u)
          for j in range(nch)]
    hi = [pltpu.async_copy(it.at[idx_i.at[pl.ds(j * _IDXW, _IDXW)]],
                           rows_i.at[pl.ds(j * _IDXW, _IDXW)], sem_i)
          for j in range(nch)]
    pltpu.sync_copy(ts.at[pl.ds(hbase, bpw)], ts_v)
    pltpu.sync_copy(dow.at[pl.ds(hbase, bpw)], dow_v)
    zeros16 = jnp.zeros((16,), jnp.float32)
    ones16 = jnp.ones((16,), jnp.float32)
    for g in range(bpw // 16):
        sl = pl.ds(g * 16, 16)
        dow16 = dow_v[sl]
        ts16 = ts_v[sl]
        for j in range(16):
            if j < 7:
                val = jnp.where(dow16 == j, ones16, zeros16)
            elif j == 8:
                val = ts16
            else:
                val = zeros16
            ebuf[j, sl] = val
    hs = [pltpu.async_copy(ebuf, et.at[:, pl.ds(base, bpw)], sem_s)]
    for j in range(nch):
        hu[j].wait()
        hs.append(pltpu.async_copy(
            rows_u.at[pl.ds(j * _IDXW, _IDXW)],
            xc.at[pl.ds(base + j * _IDXW, _IDXW), pl.ds(0, ED)], sem_s))
    for j in range(nch):
        hi[j].wait()
        hs.append(pltpu.async_copy(
            rows_i.at[pl.ds(j * _IDXW, _IDXW)],
            xc.at[pl.ds(base + j * _IDXW, _IDXW), pl.ds(ED, ED)], sem_s))
    for h in hs:
        h.wait()


@functools.cache
def _make_sc_gather(start, nrows):
    bpw = nrows // _NW
    nch = bpw // _IDXW
    return pl.kernel(
        functools.partial(_gather_body, start, bpw, nch),
        out_type=(jax.ShapeDtypeStruct((nrows, 2 * ED), jnp.float32),
                  jax.ShapeDtypeStruct((16, nrows), jnp.float32)),
        mesh=plsc.VectorSubcoreMesh(core_axis_name="c", subcore_axis_name="s"),
        scratch_types=[
            pltpu.VMEM((bpw,), jnp.int32),
            pltpu.VMEM((bpw,), jnp.int32),
            pltpu.VMEM((bpw, ED), jnp.float32),
            pltpu.VMEM((bpw, ED), jnp.float32),
            pltpu.VMEM((bpw,), jnp.float32),
            pltpu.VMEM((bpw,), jnp.int32),
            pltpu.VMEM((16, bpw), jnp.float32),
            pltpu.SemaphoreType.DMA,
            pltpu.SemaphoreType.DMA,
            pltpu.SemaphoreType.DMA,
        ],
    )


_TB = 4096


def _mlp_body(xc, e, w01, w0ext, sel16,
              b0, g0, be0, m0, v0,
              w1, b1, g1, be1, m1, v1,
              w2, b2, g2, be2, m2, v2,
              wf, bf, out):
    f32 = jnp.float32
    ew = jnp.dot(sel16[...], w0ext[...], preferred_element_type=f32)

    h = jnp.dot(xc[...], w01[...], preferred_element_type=f32)
    h += lax.dot_general(e[...], ew, (((0,), (0,)), ((), ())),
                         preferred_element_type=f32)
    s = g0[...] * lax.rsqrt(v0[...] + 1e-5)
    t = (b0[...] - m0[...]) * s + be0[...]
    h = jnp.maximum(h * s + t, 0.0)

    h = jnp.dot(h, w1[...], preferred_element_type=f32)
    s = g1[...] * lax.rsqrt(v1[...] + 1e-5)
    t = (b1[...] - m1[...]) * s + be1[...]
    h = jnp.maximum(h * s + t, 0.0)

    h = jnp.dot(h, w2[...], preferred_element_type=f32)
    s = g2[...] * lax.rsqrt(v2[...] + 1e-5)
    t = (b2[...] - m2[...]) * s + be2[...]
    h = jnp.maximum(h * s + t, 0.0)

    z = jnp.dot(h, wf[...], preferred_element_type=f32)
    z8 = jnp.reshape(z, (_TB // 128, 128)) + bf[...]
    out[...] = 5.0 / (1.0 + jnp.exp(-z8))


def _full(shape):
    return pl.BlockSpec(shape, lambda i: (0, 0))


@functools.cache
def _make_mlp(nrows):
  return pl.pallas_call(
    _mlp_body,
    grid=(nrows // _TB,),
    in_specs=[
        pl.BlockSpec((_TB, 2 * ED), lambda i: (i, 0)),
        pl.BlockSpec((16, _TB), lambda i: (0, i)),
        _full((2 * ED, 1024)),
        _full((8, 1024)),
        _full((16, 8)),
        _full((1, 1024)), _full((1, 1024)), _full((1, 1024)), _full((1, 1024)), _full((1, 1024)),
        _full((1024, 512)),
        _full((1, 512)), _full((1, 512)), _full((1, 512)), _full((1, 512)), _full((1, 512)),
        _full((512, 256)),
        _full((1, 256)), _full((1, 256)), _full((1, 256)), _full((1, 256)), _full((1, 256)),
        _full((2 * ED, 1)),
        _full((1, 1)),
    ],
    out_specs=pl.BlockSpec((_TB // 128, 128), lambda i: (i, 0)),
    out_shape=jax.ShapeDtypeStruct((nrows // 128, 128), jnp.float32),
    compiler_params=pltpu.CompilerParams(
        dimension_semantics=("parallel",),
    ),
  )


def kernel(user_ids, item_ids, timestamps, day_of_week,
           user_table, item_table, day_table,
           W0, b0, g0, be0, m0, v0,
           W1, b1, g1, be1, m1, v1,
           W2, b2, g2, be2, m2, v2,
           Wf, bf):
    uid2 = user_ids.astype(jnp.int32)
    iid2 = item_ids.astype(jnp.int32)
    dow = day_of_week.astype(jnp.int32)

    w01 = W0[:2 * ED]
    w0ext = jnp.pad(W0[2 * ED:], ((0, 3), (0, 0)))
    sel16 = (jnp.zeros((16, 8), jnp.float32)
             .at[:7, 1:5].set(day_table).at[8, 0].set(1.0))

    bn = (b0.reshape(1, -1), g0.reshape(1, -1), be0.reshape(1, -1), m0.reshape(1, -1), v0.reshape(1, -1),
          W1,
          b1.reshape(1, -1), g1.reshape(1, -1), be1.reshape(1, -1), m1.reshape(1, -1), v1.reshape(1, -1),
          W2,
          b2.reshape(1, -1), g2.reshape(1, -1), be2.reshape(1, -1), m2.reshape(1, -1), v2.reshape(1, -1),
          Wf, bf.reshape(1, 1))

    outs = []
    for start, nrows in _CHUNKS:
        xc, e_c = _make_sc_gather(start, nrows)(user_table, item_table,
                                                uid2, iid2, timestamps, dow)
        outs.append(_make_mlp(nrows)(xc, e_c, w01, w0ext, sel16, *bn))
    return jnp.concatenate(outs, axis=0).reshape(B, 1)
